# baseline (speedup 1.0000x reference)
import functools

import jax
import jax.numpy as jnp
from jax import lax
from jax.experimental import pallas as pl
from jax.experimental.pallas import tpu as pltpu
from jax.experimental.pallas import tpu_sc as plsc

NUM_CORES = 2
NUM_SUBCORES = 16
NUM_WORKERS = NUM_CORES * NUM_SUBCORES


NBUF = 4


@functools.partial(jax.jit, static_argnums=(2, 3, 4))
def _sc_gather(idx, weight, fields, chunk, nchunk):
    total = idx.shape[0]
    dim = weight.shape[1]
    batch = total // fields
    b_per_w = total // NUM_WORKERS
    xrows_per_chunk = chunk // fields
    mesh = plsc.VectorSubcoreMesh(core_axis_name="c", subcore_axis_name="s")

    @functools.partial(
        pl.kernel,
        mesh=mesh,
        out_type=jax.ShapeDtypeStruct((batch, fields, dim), jnp.float32),
        scratch_types=[
            pltpu.VMEM((b_per_w,), jnp.int32),
            [pltpu.VMEM((chunk, dim), jnp.float32) for _ in range(NBUF)],
            [pltpu.SemaphoreType.DMA for _ in range(NBUF)],
            [pltpu.SemaphoreType.DMA for _ in range(NBUF)],
        ],
        compiler_params=pltpu.CompilerParams(use_tc_tiling_on_sc=False),
    )
    def body(idx_hbm, table_hbm, out_hbm, idx_v, rows_v, gsem, ssem):
        wid = lax.axis_index("s") * NUM_CORES + lax.axis_index("c")
        base = wid * b_per_w
        xrow0 = wid * (b_per_w // fields)

        def gather(c, b):
            return pltpu.async_copy(
                table_hbm.at[idx_v.at[pl.ds(c * chunk, chunk)]], rows_v[b], gsem[b]
            )

        def store(c, b):
            @pl.loop(0, xrows_per_chunk)
            def _(r):
                pltpu.async_copy(
                    rows_v[b].at[pl.ds(r * fields, fields), :],
                    out_hbm.at[xrow0 + c * xrows_per_chunk + r],
                    ssem[b],
                )

        def drain_store(b):
            @pl.loop(0, xrows_per_chunk)
            def _(r):
                pltpu.make_async_copy(
                    out_hbm.at[0], rows_v[b].at[pl.ds(0, fields), :], ssem[b]
                ).wait()

        pltpu.sync_copy(idx_hbm.at[pl.ds(base, b_per_w)], idx_v)
        g = [None] * NBUF
        for b in range(min(NBUF, nchunk)):
            g[b] = gather(b, b)
        for c in range(nchunk):
            b = c % NBUF
            g[b].wait()
            store(c, b)
            nxt = c + NBUF
            if nxt < nchunk:
                drain_store(b)
                g[b] = gather(nxt, b)
            else:
                drain_store(b)

    return body(idx, weight)


def kernel(x, weight):
    batch, fields = x.shape
    total = batch * fields
    flat = x.reshape(total).astype(jnp.int32)
    b_per_w = total // NUM_WORKERS
    nchunk = 16
    chunk = b_per_w // nchunk
    return _sc_gather(flat, weight, fields, chunk, nchunk)


# speedup vs baseline: 1.2612x; 1.2612x over previous
import functools

import jax
import jax.numpy as jnp
from jax import lax
from jax.experimental import pallas as pl
from jax.experimental.pallas import tpu as pltpu
from jax.experimental.pallas import tpu_sc as plsc

NUM_CORES = 2
NUM_SUBCORES = 16
NUM_WORKERS = NUM_CORES * NUM_SUBCORES


NBUF = 4


@functools.partial(jax.jit, static_argnums=(2, 3, 4))
def _sc_gather(idx, weight, fields, chunk, nchunk):
    total = idx.shape[0]
    dim = weight.shape[1]
    batch = total // fields
    b_per_w = total // NUM_WORKERS
    xrows_per_chunk = chunk // fields
    mesh = plsc.VectorSubcoreMesh(core_axis_name="c", subcore_axis_name="s")

    @functools.partial(
        pl.kernel,
        mesh=mesh,
        out_type=jax.ShapeDtypeStruct((batch, 32, 128), jnp.float32),
        scratch_types=[
            pltpu.VMEM((b_per_w,), jnp.int32),
            [pltpu.VMEM((chunk, dim), jnp.float32) for _ in range(NBUF)],
            [pltpu.SemaphoreType.DMA for _ in range(NBUF)],
            [pltpu.SemaphoreType.DMA for _ in range(NBUF)],
        ],
        compiler_params=pltpu.CompilerParams(use_tc_tiling_on_sc=False),
    )
    def body(idx_hbm, table_hbm, out_hbm, idx_v, rows_v, gsem, ssem):
        wid = lax.axis_index("s") * NUM_CORES + lax.axis_index("c")
        base = wid * b_per_w
        xrow0 = wid * (b_per_w // fields)

        def gather(c, b):
            return pltpu.async_copy(
                table_hbm.at[idx_v.at[pl.ds(c * chunk, chunk)]], rows_v[b], gsem[b]
            )

        def store(c, b):
            @pl.loop(0, xrows_per_chunk)
            def _(r):
                gi = xrow0 + c * xrows_per_chunk + r
                pltpu.async_copy(
                    rows_v[b].at[pl.ds(r * fields, fields), :],
                    out_hbm.at[gi, pl.ds(0, fields), pl.ds(0, dim)],
                    ssem[b],
                )

        def drain_store(b):
            @pl.loop(0, xrows_per_chunk)
            def _(r):
                pltpu.make_async_copy(
                    out_hbm.at[0, pl.ds(0, fields), pl.ds(0, dim)],
                    rows_v[b].at[pl.ds(0, fields), :],
                    ssem[b],
                ).wait()

        pltpu.sync_copy(idx_hbm.at[pl.ds(base, b_per_w)], idx_v)
        g = [None] * NBUF
        for b in range(min(NBUF, nchunk)):
            g[b] = gather(b, b)
        for c in range(nchunk):
            b = c % NBUF
            g[b].wait()
            store(c, b)
            nxt = c + NBUF
            if nxt < nchunk:
                drain_store(b)
                g[b] = gather(nxt, b)
            else:
                drain_store(b)

    return body(idx, weight)


def kernel(x, weight):
    batch, fields = x.shape
    rows, dim = weight.shape
    total = batch * fields
    flat = x.reshape(total).astype(jnp.int32)
    b_per_w = total // NUM_WORKERS
    nchunk = 16
    chunk = b_per_w // nchunk
    out = _sc_gather(flat, weight, fields, chunk, nchunk)
    return out[:, :fields, :dim]
